# baseline (speedup 1.0000x reference)
import functools

import jax
import jax.numpy as jnp
from jax import lax
from jax.experimental import pallas as pl
from jax.experimental.pallas import tpu as pltpu
from jax.experimental.pallas import tpu_sc as plsc

VOCAB = 1000000
EMB = 64
MAXLEN = 200
BATCH = 4096

NUM_CORES = 2
NUM_SUBCORES = 16
NW = NUM_CORES * NUM_SUBCORES
ROWS_PER_W = BATCH // NW


def _make_kernel():
    mesh = plsc.VectorSubcoreMesh(core_axis_name="c", subcore_axis_name="s")

    @functools.partial(
        pl.kernel,
        mesh=mesh,
        out_type=jax.ShapeDtypeStruct((BATCH * MAXLEN, EMB), jnp.float32),
        scratch_types=[
            pltpu.VMEM((MAXLEN,), jnp.int32),
            pltpu.VMEM((MAXLEN, EMB), jnp.float32),
            pltpu.VMEM((MAXLEN, EMB), jnp.float32),
            pltpu.SemaphoreType.DMA,
        ],
        compiler_params=pltpu.CompilerParams(use_tc_tiling_on_sc=False),
    )
    def emb_kernel(x_hbm, tok_hbm, pos_hbm, out_hbm, idx_v, rows_v, pos_v, sem):
        wid = lax.axis_index("s") * NUM_CORES + lax.axis_index("c")
        pltpu.sync_copy(pos_hbm, pos_v)
        base = wid * ROWS_PER_W

        def row_body(r, carry):
            start = pl.multiple_of((base + r) * MAXLEN, 8)
            pltpu.sync_copy(x_hbm.at[pl.ds(start, MAXLEN)], idx_v)
            cp1 = pltpu.async_copy(
                tok_hbm.at[idx_v.at[pl.ds(0, 128)]], rows_v.at[pl.ds(0, 128)], sem)
            cp2 = pltpu.async_copy(
                tok_hbm.at[idx_v.at[pl.ds(128, 72)]], rows_v.at[pl.ds(128, 72)], sem)
            cp1.wait()
            cp2.wait()

            def add_body(i, c2):
                for c in range(EMB // 16):
                    sl = pl.ds(c * 16, 16)
                    rows_v[i, sl] = rows_v[i, sl] + pos_v[i, sl]
                return c2

            lax.fori_loop(0, MAXLEN, add_body, 0)
            pltpu.sync_copy(rows_v, out_hbm.at[pl.ds(start, MAXLEN)])
            return carry

        lax.fori_loop(0, ROWS_PER_W, row_body, 0)

    return emb_kernel


_EMB_KERNEL = _make_kernel()


def kernel(x, token_table, pos_table):
    x_flat = x.reshape(-1).astype(jnp.int32)
    out = _EMB_KERNEL(x_flat, token_table, pos_table)
    return out.reshape(BATCH, MAXLEN, EMB)
